# baseline (speedup 1.0000x reference)
import functools

import jax
import jax.numpy as jnp
from jax import lax
from jax.experimental import pallas as pl
from jax.experimental.pallas import tpu as pltpu
from jax.experimental.pallas import tpu_sc as plsc

_BATCH = 4
_SEQ = 8192
_HIDDEN = 1024

_NC, _NS, _L = 2, 16, 16
_NW = _NC * _NS
_POS_PER_W = _SEQ // _NW
_R = 16
_NSB = _POS_PER_W // _R


def _sc_body(x_hbm, pos_hbm, out_hbm, pbuf, xbuf):
    wid = lax.axis_index("s") * _NC + lax.axis_index("c")
    pos0 = wid * _POS_PER_W

    def sub_block(r, carry):
        base = pos0 + r * _R
        pltpu.sync_copy(pos_hbm.at[pl.ds(base, _R)], pbuf)
        for b in range(_BATCH):
            pltpu.sync_copy(x_hbm.at[b, pl.ds(base, _R)], xbuf.at[b])
        for b in range(_BATCH):
            for row in range(_R):
                def chunk(j, c, b=b, row=row):
                    for u in range(8):
                        off = (j * 8 + u) * _L
                        p = pbuf[row, pl.ds(off, _L)]
                        plsc.addupdate(xbuf.at[b, row, pl.ds(off, _L)], p)
                    return c
                lax.fori_loop(0, _HIDDEN // _L // 8, chunk, 0)
        for b in range(_BATCH):
            pltpu.sync_copy(xbuf.at[b], out_hbm.at[b, pl.ds(base, _R)])
        return carry

    lax.fori_loop(0, _NSB, sub_block, 0)


_sc_kernel = functools.partial(
    pl.kernel,
    out_type=jax.ShapeDtypeStruct((_BATCH, _SEQ, _HIDDEN), jnp.float32),
    mesh=plsc.VectorSubcoreMesh(
        core_axis_name="c", subcore_axis_name="s", num_cores=_NC, num_subcores=_NS
    ),
    scratch_types=[
        pltpu.VMEM((_R, _HIDDEN), jnp.float32),
        pltpu.VMEM((_BATCH, _R, _HIDDEN), jnp.float32),
    ],
)(_sc_body)


def kernel(x, pos_embedding):
    return _sc_kernel(x, pos_embedding)


# speedup vs baseline: 1.4272x; 1.4272x over previous
import functools

import jax
import jax.numpy as jnp
from jax import lax
from jax.experimental import pallas as pl
from jax.experimental.pallas import tpu as pltpu
from jax.experimental.pallas import tpu_sc as plsc

_BATCH = 4
_SEQ = 8192
_HIDDEN = 1024

_NC, _NS, _L = 2, 16, 16
_NW = _NC * _NS
_POS_PER_W = _SEQ // _NW
_R = 8
_NSB = _POS_PER_W // _R


def _sc_body(x_hbm, pos_hbm, out_hbm, pbuf, xbuf, sin0, sin1, sout0, sout1):
    wid = lax.axis_index("s") * _NC + lax.axis_index("c")
    pos0 = wid * _POS_PER_W
    sins = (sin0, sin1)
    souts = (sout0, sout1)

    def start_in(slot, base, sem):
        pltpu.async_copy(pos_hbm.at[pl.ds(base, _R)], pbuf.at[slot], sem)
        for b in range(_BATCH):
            pltpu.async_copy(x_hbm.at[b, pl.ds(base, _R)], xbuf.at[slot, b], sem)

    def wait_in(slot, base, sem):
        pltpu.make_async_copy(pos_hbm.at[pl.ds(base, _R)], pbuf.at[slot], sem).wait()
        for b in range(_BATCH):
            pltpu.make_async_copy(
                x_hbm.at[b, pl.ds(base, _R)], xbuf.at[slot, b], sem
            ).wait()

    def start_out(slot, base, sem):
        for b in range(_BATCH):
            pltpu.async_copy(xbuf.at[slot, b], out_hbm.at[b, pl.ds(base, _R)], sem)

    def wait_out(slot, base, sem):
        for b in range(_BATCH):
            pltpu.make_async_copy(
                xbuf.at[slot, b], out_hbm.at[b, pl.ds(base, _R)], sem
            ).wait()

    def compute(slot):
        for b in range(_BATCH):
            for row in range(_R):
                def chunk(j, c, b=b, row=row):
                    for u in range(8):
                        off = (j * 8 + u) * _L
                        p = pbuf[slot, row, pl.ds(off, _L)]
                        plsc.addupdate(xbuf.at[slot, b, row, pl.ds(off, _L)], p)
                    return c
                lax.fori_loop(0, _HIDDEN // _L // 8, chunk, 0)

    start_in(0, pos0, sins[0])
    start_in(1, pos0 + _R, sins[1])

    def step(i, carry):
        for slot in range(2):
            r = i * 2 + slot
            base = pos0 + r * _R
            wait_in(slot, base, sins[slot])
            compute(slot)
            start_out(slot, base, souts[slot])

            @pl.when(r + 2 < _NSB)
            def _():
                wait_out(slot, base, souts[slot])
                start_in(slot, base + 2 * _R, sins[slot])

        return carry

    lax.fori_loop(0, _NSB // 2, step, 0)

    for slot in range(2):
        base = pos0 + (_NSB - 2 + slot) * _R
        wait_out(slot, base, souts[slot])


_sc_kernel = functools.partial(
    pl.kernel,
    out_type=jax.ShapeDtypeStruct((_BATCH, _SEQ, _HIDDEN), jnp.float32),
    mesh=plsc.VectorSubcoreMesh(
        core_axis_name="c", subcore_axis_name="s", num_cores=_NC, num_subcores=_NS
    ),
    scratch_types=[
        pltpu.VMEM((2, _R, _HIDDEN), jnp.float32),
        pltpu.VMEM((2, _BATCH, _R, _HIDDEN), jnp.float32),
        pltpu.SemaphoreType.DMA,
        pltpu.SemaphoreType.DMA,
        pltpu.SemaphoreType.DMA,
        pltpu.SemaphoreType.DMA,
    ],
)(_sc_body)


def kernel(x, pos_embedding):
    return _sc_kernel(x, pos_embedding)


# speedup vs baseline: 2.0653x; 1.4471x over previous
import functools

import jax
import jax.numpy as jnp
from jax import lax
from jax.experimental import pallas as pl
from jax.experimental.pallas import tpu as pltpu
from jax.experimental.pallas import tpu_sc as plsc

_BATCH = 4
_SEQ = 8192
_HIDDEN = 1024

_NC, _NS, _L = 2, 16, 16
_NW = _NC * _NS
_POS_PER_W = _SEQ // _NW
_R = 8
_NSB = _POS_PER_W // _R


def _sc_body(x_hbm, pos_hbm, out_hbm, pbuf, xbuf, sin0, sin1, sout0, sout1):
    wid = lax.axis_index("s") * _NC + lax.axis_index("c")
    pos0 = wid * _POS_PER_W
    sins = (sin0, sin1)
    souts = (sout0, sout1)

    def start_in(slot, base, sem):
        pltpu.async_copy(pos_hbm.at[pl.ds(base, _R)], pbuf.at[slot], sem)
        for b in range(_BATCH):
            pltpu.async_copy(x_hbm.at[b, pl.ds(base, _R)], xbuf.at[slot, b], sem)

    def wait_in(slot, base, sem):
        pltpu.make_async_copy(pos_hbm.at[pl.ds(base, _R)], pbuf.at[slot], sem).wait()
        for b in range(_BATCH):
            pltpu.make_async_copy(
                x_hbm.at[b, pl.ds(base, _R)], xbuf.at[slot, b], sem
            ).wait()

    def start_out(slot, base, sem):
        for b in range(_BATCH):
            pltpu.async_copy(xbuf.at[slot, b], out_hbm.at[b, pl.ds(base, _R)], sem)

    def wait_out(slot, base, sem):
        for b in range(_BATCH):
            pltpu.make_async_copy(
                xbuf.at[slot, b], out_hbm.at[b, pl.ds(base, _R)], sem
            ).wait()

    def compute(slot):
        def quarter(j, c):
            for b in range(_BATCH):
                for row in range(_R):
                    for u in range(16):
                        off = (j * 16 + u) * _L
                        p = pbuf[slot, row, pl.ds(off, _L)]
                        plsc.addupdate(xbuf.at[slot, b, row, pl.ds(off, _L)], p)
            return c
        lax.fori_loop(0, _HIDDEN // _L // 16, quarter, 0)

    start_in(0, pos0, sins[0])
    start_in(1, pos0 + _R, sins[1])

    def step(i, carry):
        for slot in range(2):
            r = i * 2 + slot
            base = pos0 + r * _R
            wait_in(slot, base, sins[slot])
            compute(slot)
            start_out(slot, base, souts[slot])

            @pl.when(r + 2 < _NSB)
            def _():
                wait_out(slot, base, souts[slot])
                start_in(slot, base + 2 * _R, sins[slot])

        return carry

    lax.fori_loop(0, _NSB // 2, step, 0)

    for slot in range(2):
        base = pos0 + (_NSB - 2 + slot) * _R
        wait_out(slot, base, souts[slot])


_sc_kernel = functools.partial(
    pl.kernel,
    out_type=jax.ShapeDtypeStruct((_BATCH, _SEQ, _HIDDEN), jnp.float32),
    mesh=plsc.VectorSubcoreMesh(
        core_axis_name="c", subcore_axis_name="s", num_cores=_NC, num_subcores=_NS
    ),
    scratch_types=[
        pltpu.VMEM((2, _R, _HIDDEN), jnp.float32),
        pltpu.VMEM((2, _BATCH, _R, _HIDDEN), jnp.float32),
        pltpu.SemaphoreType.DMA,
        pltpu.SemaphoreType.DMA,
        pltpu.SemaphoreType.DMA,
        pltpu.SemaphoreType.DMA,
    ],
)(_sc_body)


def kernel(x, pos_embedding):
    return _sc_kernel(x, pos_embedding)


# speedup vs baseline: 2.1838x; 1.0574x over previous
import functools

import jax
import jax.numpy as jnp
from jax import lax
from jax.experimental import pallas as pl
from jax.experimental.pallas import tpu as pltpu
from jax.experimental.pallas import tpu_sc as plsc

_BATCH = 4
_SEQ = 8192
_HIDDEN = 1024

_NC, _NS, _L = 2, 16, 16
_NW = _NC * _NS
_POS_PER_W = _SEQ // _NW
_R = 8
_NSB = _POS_PER_W // _R
_NSLOT = 3


def _sc_body(x_hbm, pos_hbm, out_hbm, pbuf, xbuf, *sems):
    sins = sems[:_NSLOT]
    souts = sems[_NSLOT:]
    wid = lax.axis_index("s") * _NC + lax.axis_index("c")
    pos0 = wid * _POS_PER_W

    def start_in(slot, base):
        pltpu.async_copy(pos_hbm.at[pl.ds(base, _R)], pbuf.at[slot], sins[slot])
        for b in range(_BATCH):
            pltpu.async_copy(x_hbm.at[b, pl.ds(base, _R)], xbuf.at[slot, b], sins[slot])

    def wait_in(slot, base):
        pltpu.make_async_copy(pos_hbm.at[pl.ds(base, _R)], pbuf.at[slot], sins[slot]).wait()
        for b in range(_BATCH):
            pltpu.make_async_copy(
                x_hbm.at[b, pl.ds(base, _R)], xbuf.at[slot, b], sins[slot]
            ).wait()

    def start_out(slot, base):
        for b in range(_BATCH):
            pltpu.async_copy(xbuf.at[slot, b], out_hbm.at[b, pl.ds(base, _R)], souts[slot])

    def wait_out(slot, base):
        for b in range(_BATCH):
            pltpu.make_async_copy(
                xbuf.at[slot, b], out_hbm.at[b, pl.ds(base, _R)], souts[slot]
            ).wait()

    def compute(slot):
        def quarter(j, c):
            for row in range(_R):
                for u in range(16):
                    off = (j * 16 + u) * _L
                    p = pbuf[slot, row, pl.ds(off, _L)]
                    for b in range(_BATCH):
                        plsc.addupdate(xbuf.at[slot, b, row, pl.ds(off, _L)], p)
            return c
        lax.fori_loop(0, _HIDDEN // _L // 16, quarter, 0)

    def process(r, slot, first, last):
        base = pos0 + r * _R
        wait_in(slot, base)
        compute(slot)
        start_out(slot, base)
        if not last:
            nxt_slot = (slot + 2) % _NSLOT
            if first:
                start_in(nxt_slot, base + 2 * _R)
            else:
                @pl.when(r + 2 < _NSB)
                def _():
                    wait_out(nxt_slot, base - _R)
                    start_in(nxt_slot, base + 2 * _R)

    start_in(0, pos0)
    start_in(1, pos0 + _R)

    process(0, 0, first=True, last=False)

    def step(i, carry):
        for s_off in range(_NSLOT):
            r = 1 + i * _NSLOT + s_off
            process(r, (1 + s_off) % _NSLOT, first=False, last=False)
        return carry

    lax.fori_loop(0, (_NSB - 2) // _NSLOT, step, 0)

    process(_NSB - 1, (_NSB - 1) % _NSLOT, first=False, last=True)

    for r in (_NSB - 3, _NSB - 2, _NSB - 1):
        wait_out(r % _NSLOT, pos0 + r * _R)


_sc_kernel = functools.partial(
    pl.kernel,
    out_type=jax.ShapeDtypeStruct((_BATCH, _SEQ, _HIDDEN), jnp.float32),
    mesh=plsc.VectorSubcoreMesh(
        core_axis_name="c", subcore_axis_name="s", num_cores=_NC, num_subcores=_NS
    ),
    scratch_types=[
        pltpu.VMEM((_NSLOT, _R, _HIDDEN), jnp.float32),
        pltpu.VMEM((_NSLOT, _BATCH, _R, _HIDDEN), jnp.float32),
    ]
    + [pltpu.SemaphoreType.DMA] * (2 * _NSLOT),
)(_sc_body)


def kernel(x, pos_embedding):
    return _sc_kernel(x, pos_embedding)


# speedup vs baseline: 2.2001x; 1.0075x over previous
import functools

import jax
import jax.numpy as jnp
from jax import lax
from jax.experimental import pallas as pl
from jax.experimental.pallas import tpu as pltpu
from jax.experimental.pallas import tpu_sc as plsc

_BATCH = 4
_SEQ = 8192
_HIDDEN = 1024

_NC, _NS, _L = 2, 16, 16
_NW = _NC * _NS
_POS_PER_W = _SEQ // _NW
_R = 4
_NSB = _POS_PER_W // _R
_NSLOT = 4


def _sc_body(x_hbm, pos_hbm, out_hbm, pbuf, xbuf, *sems):
    sins = sems[:_NSLOT]
    souts = sems[_NSLOT:]
    wid = lax.axis_index("s") * _NC + lax.axis_index("c")
    pos0 = wid * _POS_PER_W

    def start_in(slot, base):
        pltpu.async_copy(pos_hbm.at[pl.ds(base, _R)], pbuf.at[slot], sins[slot])
        for b in range(_BATCH):
            pltpu.async_copy(x_hbm.at[b, pl.ds(base, _R)], xbuf.at[slot, b], sins[slot])

    def wait_in(slot, base):
        pltpu.make_async_copy(pos_hbm.at[pl.ds(base, _R)], pbuf.at[slot], sins[slot]).wait()
        for b in range(_BATCH):
            pltpu.make_async_copy(
                x_hbm.at[b, pl.ds(base, _R)], xbuf.at[slot, b], sins[slot]
            ).wait()

    def start_out(slot, base):
        for b in range(_BATCH):
            pltpu.async_copy(xbuf.at[slot, b], out_hbm.at[b, pl.ds(base, _R)], souts[slot])

    def wait_out(slot, base):
        for b in range(_BATCH):
            pltpu.make_async_copy(
                xbuf.at[slot, b], out_hbm.at[b, pl.ds(base, _R)], souts[slot]
            ).wait()

    def compute(slot):
        def quarter(j, c):
            for row in range(_R):
                for u in range(16):
                    off = (j * 16 + u) * _L
                    p = pbuf[slot, row, pl.ds(off, _L)]
                    for b in range(_BATCH):
                        plsc.addupdate(xbuf.at[slot, b, row, pl.ds(off, _L)], p)
            return c
        lax.fori_loop(0, _HIDDEN // _L // 16, quarter, 0)

    def process(r, slot, first, last):
        base = pos0 + r * _R
        wait_in(slot, base)
        compute(slot)
        start_out(slot, base)
        if not last:
            nxt_slot = (slot + 2) % _NSLOT
            if first:
                start_in(nxt_slot, base + 2 * _R)
            else:
                @pl.when(r + 2 < _NSB)
                def _():
                    wait_out(nxt_slot, base - (_NSLOT - 2) * _R)
                    start_in(nxt_slot, base + 2 * _R)

    start_in(0, pos0)
    start_in(1, pos0 + _R)

    for r in range(_NSLOT - 2):
        process(r, r % _NSLOT, first=True, last=False)

    _head = _NSLOT - 2
    _main = ((_NSB - _head) // _NSLOT) * _NSLOT

    def step(i, carry):
        for s_off in range(_NSLOT):
            r = _head + i * _NSLOT + s_off
            process(r, (_head + s_off) % _NSLOT, first=False, last=False)
        return carry

    lax.fori_loop(0, _main // _NSLOT, step, 0)

    for r in range(_head + _main, _NSB):
        process(r, r % _NSLOT, first=False, last=True)

    for r in range(_NSB - _NSLOT, _NSB):
        wait_out(r % _NSLOT, pos0 + r * _R)


_sc_kernel = functools.partial(
    pl.kernel,
    out_type=jax.ShapeDtypeStruct((_BATCH, _SEQ, _HIDDEN), jnp.float32),
    mesh=plsc.VectorSubcoreMesh(
        core_axis_name="c", subcore_axis_name="s", num_cores=_NC, num_subcores=_NS
    ),
    scratch_types=[
        pltpu.VMEM((_NSLOT, _R, _HIDDEN), jnp.float32),
        pltpu.VMEM((_NSLOT, _BATCH, _R, _HIDDEN), jnp.float32),
    ]
    + [pltpu.SemaphoreType.DMA] * (2 * _NSLOT),
)(_sc_body)


def kernel(x, pos_embedding):
    return _sc_kernel(x, pos_embedding)


# speedup vs baseline: 2.3421x; 1.0646x over previous
import functools

import jax
import jax.numpy as jnp
from jax import lax
from jax.experimental import pallas as pl
from jax.experimental.pallas import tpu as pltpu
from jax.experimental.pallas import tpu_sc as plsc

_BATCH = 4
_SEQ = 8192
_HIDDEN = 1024

_NC, _NS, _L = 2, 16, 16
_NW = _NC * _NS
_POS_PER_W = _SEQ // _NW
_R = 4
_NSB = _POS_PER_W // _R
_NSLOT = 4


def _sc_body(x_hbm, pos_hbm, out_hbm, pbuf, xbuf, *sems):
    sins = sems[:_NSLOT]
    souts = sems[_NSLOT:]
    wid = lax.axis_index("s") * _NC + lax.axis_index("c")
    pos0 = wid * _POS_PER_W

    def start_in(slot, base):
        pltpu.async_copy(pos_hbm.at[pl.ds(base, _R)], pbuf.at[slot], sins[slot])
        for b in range(_BATCH):
            pltpu.async_copy(x_hbm.at[b, pl.ds(base, _R)], xbuf.at[slot, b], sins[slot])

    def wait_in(slot, base):
        pltpu.make_async_copy(pos_hbm.at[pl.ds(base, _R)], pbuf.at[slot], sins[slot]).wait()
        for b in range(_BATCH):
            pltpu.make_async_copy(
                x_hbm.at[b, pl.ds(base, _R)], xbuf.at[slot, b], sins[slot]
            ).wait()

    def start_out(slot, base):
        for b in range(_BATCH):
            pltpu.async_copy(xbuf.at[slot, b], out_hbm.at[b, pl.ds(base, _R)], souts[slot])

    def wait_out(slot, base):
        for b in range(_BATCH):
            pltpu.make_async_copy(
                xbuf.at[slot, b], out_hbm.at[b, pl.ds(base, _R)], souts[slot]
            ).wait()

    def compute(slot):
        def quarter(j, c):
            for row in range(_R):
                for u in range(16):
                    off = (j * 16 + u) * _L
                    p = pbuf[slot, row, pl.ds(off, _L)]
                    for b in range(_BATCH):
                        plsc.addupdate(xbuf.at[slot, b, row, pl.ds(off, _L)], p)
            return c
        lax.fori_loop(0, _HIDDEN // _L // 16, quarter, 0)

    def process(r, slot, first, last):
        base = pos0 + r * _R
        wait_in(slot, base)
        if not last:
            nxt_slot = (slot + 2) % _NSLOT
            if first:
                start_in(nxt_slot, base + 2 * _R)
            else:
                @pl.when(r + 2 < _NSB)
                def _():
                    wait_out(nxt_slot, base - (_NSLOT - 2) * _R)
                    start_in(nxt_slot, base + 2 * _R)
        compute(slot)
        start_out(slot, base)

    start_in(0, pos0)
    start_in(1, pos0 + _R)

    for r in range(_NSLOT - 2):
        process(r, r % _NSLOT, first=True, last=False)

    _head = _NSLOT - 2
    _main = ((_NSB - _head) // _NSLOT) * _NSLOT

    def step(i, carry):
        for s_off in range(_NSLOT):
            r = _head + i * _NSLOT + s_off
            process(r, (_head + s_off) % _NSLOT, first=False, last=False)
        return carry

    lax.fori_loop(0, _main // _NSLOT, step, 0)

    for r in range(_head + _main, _NSB):
        process(r, r % _NSLOT, first=False, last=True)

    for r in range(_NSB - _NSLOT, _NSB):
        wait_out(r % _NSLOT, pos0 + r * _R)


_sc_kernel = functools.partial(
    pl.kernel,
    out_type=jax.ShapeDtypeStruct((_BATCH, _SEQ, _HIDDEN), jnp.float32),
    mesh=plsc.VectorSubcoreMesh(
        core_axis_name="c", subcore_axis_name="s", num_cores=_NC, num_subcores=_NS
    ),
    scratch_types=[
        pltpu.VMEM((_NSLOT, _R, _HIDDEN), jnp.float32),
        pltpu.VMEM((_NSLOT, _BATCH, _R, _HIDDEN), jnp.float32),
    ]
    + [pltpu.SemaphoreType.DMA] * (2 * _NSLOT),
)(_sc_body)


def kernel(x, pos_embedding):
    return _sc_kernel(x, pos_embedding)


# speedup vs baseline: 2.3647x; 1.0096x over previous
import functools

import jax
import jax.numpy as jnp
from jax import lax
from jax.experimental import pallas as pl
from jax.experimental.pallas import tpu as pltpu
from jax.experimental.pallas import tpu_sc as plsc

_BATCH = 4
_SEQ = 8192
_HIDDEN = 1024

_NC, _NS, _L = 2, 16, 16
_NW = _NC * _NS
_POS_PER_W = _SEQ // _NW
_R = 4
_NSB = _POS_PER_W // _R
_NSLOT = 4


def _sc_body(x_hbm, pos_hbm, out_hbm, pbuf, xbuf, *sems):
    sins = sems[:_NSLOT]
    souts = sems[_NSLOT:]
    wid = lax.axis_index("s") * _NC + lax.axis_index("c")
    pos0 = wid * _POS_PER_W

    def start_in(slot, base):
        pltpu.async_copy(pos_hbm.at[pl.ds(base, _R)], pbuf.at[slot], sins[slot])
        pltpu.async_copy(x_hbm.at[:, pl.ds(base, _R)], xbuf.at[slot], sins[slot])

    def wait_in(slot, base):
        pltpu.make_async_copy(pos_hbm.at[pl.ds(base, _R)], pbuf.at[slot], sins[slot]).wait()
        pltpu.make_async_copy(
            x_hbm.at[:, pl.ds(base, _R)], xbuf.at[slot], sins[slot]
        ).wait()

    def start_out(slot, base):
        pltpu.async_copy(xbuf.at[slot], out_hbm.at[:, pl.ds(base, _R)], souts[slot])

    def wait_out(slot, base):
        pltpu.make_async_copy(
            xbuf.at[slot], out_hbm.at[:, pl.ds(base, _R)], souts[slot]
        ).wait()

    def compute(slot):
        def quarter(j, c):
            for row in range(_R):
                for u in range(16):
                    off = (j * 16 + u) * _L
                    p = pbuf[slot, row, pl.ds(off, _L)]
                    for b in range(_BATCH):
                        plsc.addupdate(xbuf.at[slot, b, row, pl.ds(off, _L)], p)
            return c
        lax.fori_loop(0, _HIDDEN // _L // 16, quarter, 0)

    def process(r, slot, first, last):
        base = pos0 + r * _R
        wait_in(slot, base)
        if not last:
            nxt_slot = (slot + 2) % _NSLOT
            if first:
                start_in(nxt_slot, base + 2 * _R)
            else:
                @pl.when(r + 2 < _NSB)
                def _():
                    wait_out(nxt_slot, base - (_NSLOT - 2) * _R)
                    start_in(nxt_slot, base + 2 * _R)
        compute(slot)
        start_out(slot, base)

    start_in(0, pos0)
    start_in(1, pos0 + _R)

    for r in range(_NSLOT - 2):
        process(r, r % _NSLOT, first=True, last=False)

    _head = _NSLOT - 2
    _main = ((_NSB - _head) // _NSLOT) * _NSLOT

    def step(i, carry):
        for s_off in range(_NSLOT):
            r = _head + i * _NSLOT + s_off
            process(r, (_head + s_off) % _NSLOT, first=False, last=False)
        return carry

    lax.fori_loop(0, _main // _NSLOT, step, 0)

    for r in range(_head + _main, _NSB):
        process(r, r % _NSLOT, first=False, last=True)

    for r in range(_NSB - _NSLOT, _NSB):
        wait_out(r % _NSLOT, pos0 + r * _R)


_sc_kernel = functools.partial(
    pl.kernel,
    out_type=jax.ShapeDtypeStruct((_BATCH, _SEQ, _HIDDEN), jnp.float32),
    mesh=plsc.VectorSubcoreMesh(
        core_axis_name="c", subcore_axis_name="s", num_cores=_NC, num_subcores=_NS
    ),
    scratch_types=[
        pltpu.VMEM((_NSLOT, _R, _HIDDEN), jnp.float32),
        pltpu.VMEM((_NSLOT, _BATCH, _R, _HIDDEN), jnp.float32),
    ]
    + [pltpu.SemaphoreType.DMA] * (2 * _NSLOT),
)(_sc_body)


def kernel(x, pos_embedding):
    return _sc_kernel(x, pos_embedding)
